# Initial kernel scaffold; baseline (speedup 1.0000x reference)
import jax
import jax.numpy as jnp
from jax.experimental import pallas as pl


def kernel(node_embeddings, gc_weights, edge_index):
    raise NotImplementedError("write your pallas kernel here")


# speedup vs baseline: 4.3850x; 4.3850x over previous
import functools

import jax
import jax.numpy as jnp
from jax import lax
from jax.experimental import pallas as pl
from jax.experimental.pallas import tpu as pltpu
from jax.experimental.pallas import tpu_sc as plsc

NC = 2
NS = 16
CHUNK = 128
AUG = 16


def _sc_scatter(n_pad, w, e_pad, rows_per_tile, steps_per_tile):
    edges_per_tile = e_pad // (NC * NS)
    row_chunks = rows_per_tile // CHUNK
    mesh = plsc.VectorSubcoreMesh(core_axis_name="c", subcore_axis_name="s")

    @functools.partial(
        pl.kernel,
        mesh=mesh,
        compiler_params=pltpu.CompilerParams(use_tc_tiling_on_sc=False),
        out_type=jax.ShapeDtypeStruct((NC * n_pad, w), jnp.float32),
        scratch_types=[
            pltpu.VMEM_SHARED((n_pad, w), jnp.float32),
            pltpu.VMEM((CHUNK,), jnp.int32),
            pltpu.VMEM((CHUNK,), jnp.int32),
            pltpu.VMEM((CHUNK, w), jnp.float32),
            pltpu.SemaphoreType.DMA,
        ],
    )
    def sc_kernel(xa_hbm, src_hbm, dst_hbm, z_hbm,
                  psum_hbm,
                  acc, sidx, didx, rows, sem):
        c = lax.axis_index("c")
        s = lax.axis_index("s")
        t = c * NS + s
        row0 = s * rows_per_tile

        pltpu.sync_copy(z_hbm, rows)
        for j in range(row_chunks):
            pltpu.sync_copy(rows, acc.at[pl.ds(row0 + j * CHUNK, CHUNK)])
        plsc.subcore_barrier()

        ebase = t * edges_per_tile

        @pl.loop(0, steps_per_tile)
        def step(i):
            base = pl.multiple_of(ebase + i * CHUNK, CHUNK)
            pltpu.sync_copy(src_hbm.at[pl.ds(base, CHUNK)], sidx)
            pltpu.sync_copy(dst_hbm.at[pl.ds(base, CHUNK)], didx)
            pltpu.async_copy(xa_hbm.at[sidx], rows, sem).wait()
            pltpu.sync_copy(rows, acc.at[didx], add=True)

        plsc.subcore_barrier()

        out0 = c * n_pad + row0
        for j in range(row_chunks):
            pltpu.sync_copy(acc.at[pl.ds(row0 + j * CHUNK, CHUNK)], rows)
            pltpu.sync_copy(rows, psum_hbm.at[pl.ds(out0 + j * CHUNK, CHUNK)])

    return sc_kernel


def _tc_combine(n_pad, d, w, blk):
    def body(p_ref, w_ref, o_ref):
        ssum = p_ref[0] + p_ref[1]
        deg = ssum[:, d:d + 1]
        inv = 1.0 / jnp.maximum(deg, 1.0)
        o_ref[...] = jnp.dot(ssum[:, :d] * inv, w_ref[...],
                             preferred_element_type=jnp.float32)

    return pl.pallas_call(
        body,
        grid=(n_pad // blk,),
        in_specs=[
            pl.BlockSpec((NC, blk, w), lambda i: (0, i, 0)),
            pl.BlockSpec((d, d), lambda i: (0, 0)),
        ],
        out_specs=pl.BlockSpec((blk, d), lambda i: (i, 0)),
        out_shape=jax.ShapeDtypeStruct((n_pad, d), jnp.float32),
    )


def kernel(node_embeddings, gc_weights, edge_index):
    n, d = node_embeddings.shape
    e = edge_index.shape[1]
    w = d + AUG

    step_edges = NC * NS * CHUNK
    e_pad = ((e + step_edges - 1) // step_edges) * step_edges
    steps_per_tile = e_pad // step_edges
    blk = 512
    quantum = NS * CHUNK
    n_pad = ((n + 1 + quantum - 1) // quantum) * quantum

    rows_per_tile = n_pad // NS

    pad = e_pad - e
    src = jnp.concatenate([edge_index[0], jnp.zeros((pad,), jnp.int32)])
    dst = jnp.concatenate([edge_index[1], jnp.full((pad,), n, jnp.int32)])
    xa = jnp.concatenate(
        [node_embeddings, jnp.ones((n, AUG), jnp.float32)], axis=1)
    z = jnp.zeros((CHUNK, w), jnp.float32)

    psum = _sc_scatter(n_pad, w, e_pad, rows_per_tile, steps_per_tile)(
        xa, src, dst, z)
    psum = psum.reshape(NC, n_pad, w)
    out = _tc_combine(n_pad, d, w, blk)(psum, gc_weights)
    return out[:n]


# speedup vs baseline: 6.0381x; 1.3770x over previous
import functools

import jax
import jax.numpy as jnp
from jax import lax
from jax.experimental import pallas as pl
from jax.experimental.pallas import tpu as pltpu
from jax.experimental.pallas import tpu_sc as plsc

NC = 2
NS = 16
CHUNK = 64
DW = 16


def _sc_scatter(n_pad, d, e_pad, rows_per_tile, steps_per_tile):
    row_chunks = rows_per_tile // CHUNK
    mesh = plsc.VectorSubcoreMesh(core_axis_name="c", subcore_axis_name="s")

    nbuf = 2

    @functools.partial(
        pl.kernel,
        mesh=mesh,
        compiler_params=pltpu.CompilerParams(use_tc_tiling_on_sc=False),
        out_type=[
            jax.ShapeDtypeStruct((NC * n_pad, d), jnp.float32),
            jax.ShapeDtypeStruct((NC * n_pad, DW), jnp.float32),
        ],
        scratch_types=[
            pltpu.VMEM_SHARED((n_pad, d), jnp.float32),
            pltpu.VMEM_SHARED((n_pad, DW), jnp.float32),
            pltpu.VMEM((steps_per_tile, CHUNK), jnp.int32),
            pltpu.VMEM((steps_per_tile, CHUNK), jnp.int32),
            [pltpu.VMEM((CHUNK, d), jnp.float32)] * nbuf,
            pltpu.VMEM((CHUNK, DW), jnp.float32),
            [pltpu.SemaphoreType.DMA] * nbuf,
            [pltpu.SemaphoreType.DMA] * nbuf,
            [pltpu.SemaphoreType.DMA] * nbuf,
        ],
    )
    def sc_kernel(x_hbm, src_hbm, dst_hbm, z_hbm, z16_hbm, ones_hbm,
                  psum_hbm, pdeg_hbm,
                  acc, dacc, sidx, didx, rows, onesb, gsem, ssem, dsem):
        c = lax.axis_index("c")
        s = lax.axis_index("s")
        t = c * NS + s
        row0 = s * rows_per_tile

        pltpu.sync_copy(src_hbm.at[pl.ds(t * steps_per_tile, steps_per_tile)],
                        sidx)
        pltpu.sync_copy(dst_hbm.at[pl.ds(t * steps_per_tile, steps_per_tile)],
                        didx)
        pltpu.sync_copy(z_hbm, rows[0])
        pltpu.sync_copy(z16_hbm, onesb)
        zcps = []
        for j in range(row_chunks):
            zcps.append(pltpu.async_copy(
                rows[0], acc.at[pl.ds(row0 + j * CHUNK, CHUNK)],
                ssem[j % nbuf]))
            zcps.append(pltpu.async_copy(
                onesb, dacc.at[pl.ds(row0 + j * CHUNK, CHUNK)],
                dsem[j % nbuf]))
        for cp in zcps:
            cp.wait()
        pltpu.sync_copy(ones_hbm, onesb)
        plsc.subcore_barrier()

        def start_gather(b, i):
            return pltpu.async_copy(x_hbm.at[sidx.at[i]], rows[b], gsem[b])

        def wait_gather(b, i):
            pltpu.make_async_copy(x_hbm.at[sidx.at[i]], rows[b],
                                  gsem[b]).wait()

        def start_scatter(b, i):
            pltpu.async_copy(rows[b], acc.at[didx.at[i]], ssem[b], add=True)
            pltpu.async_copy(onesb, dacc.at[didx.at[i]], dsem[b], add=True)

        def wait_scatter(b, i):
            pltpu.make_async_copy(rows[b], acc.at[didx.at[i]],
                                  ssem[b]).wait()
            pltpu.make_async_copy(onesb, dacc.at[didx.at[i]],
                                  dsem[b]).wait()

        for b in range(nbuf):
            start_gather(b, b)

        @pl.loop(0, steps_per_tile // nbuf - 1)
        def step(ii):
            i = pl.multiple_of(ii * nbuf, nbuf)
            for b in range(nbuf):
                wait_gather(b, i + b)
                start_scatter(b, i + b)
            for b in range(nbuf):
                wait_scatter(b, i + b)
                start_gather(b, i + b + nbuf)

        i_last = steps_per_tile - nbuf
        for b in range(nbuf):
            wait_gather(b, i_last + b)
            start_scatter(b, i_last + b)
        for b in range(nbuf):
            wait_scatter(b, i_last + b)
        plsc.subcore_barrier()

        out0 = c * n_pad + row0
        for j in range(row_chunks):
            b = j % nbuf
            if j >= nbuf:
                pltpu.make_async_copy(
                    rows[b], psum_hbm.at[pl.ds(out0 + (j - nbuf) * CHUNK,
                                               CHUNK)], ssem[b]).wait()
            pltpu.sync_copy(acc.at[pl.ds(row0 + j * CHUNK, CHUNK)], rows[b])
            pltpu.async_copy(rows[b],
                             psum_hbm.at[pl.ds(out0 + j * CHUNK, CHUNK)],
                             ssem[b])
        for j in range(row_chunks - nbuf, row_chunks):
            b = j % nbuf
            pltpu.make_async_copy(
                rows[b], psum_hbm.at[pl.ds(out0 + j * CHUNK, CHUNK)],
                ssem[b]).wait()
        for j in range(row_chunks):
            pltpu.sync_copy(dacc.at[pl.ds(row0 + j * CHUNK, CHUNK)], onesb)
            pltpu.sync_copy(onesb,
                            pdeg_hbm.at[pl.ds(out0 + j * CHUNK, CHUNK)])

    return sc_kernel


def _tc_combine(n, n_pad, d, blk):
    def body(p0_ref, p1_ref, d0_ref, d1_ref, w_ref, o_ref):
        ssum = p0_ref[...] + p1_ref[...]
        deg = d0_ref[:, 0:1] + d1_ref[:, 0:1]
        inv = 1.0 / jnp.maximum(deg, 1.0)
        o_ref[...] = jnp.dot(ssum * inv, w_ref[...],
                             preferred_element_type=jnp.float32)

    nblk = n_pad // blk
    return pl.pallas_call(
        body,
        grid=(nblk,),
        in_specs=[
            pl.BlockSpec((blk, d), lambda i: (i, 0)),
            pl.BlockSpec((blk, d), lambda i, nblk=nblk: (nblk + i, 0)),
            pl.BlockSpec((blk, DW), lambda i: (i, 0)),
            pl.BlockSpec((blk, DW), lambda i, nblk=nblk: (nblk + i, 0)),
            pl.BlockSpec((d, d), lambda i: (0, 0)),
        ],
        out_specs=pl.BlockSpec((blk, d), lambda i: (i, 0)),
        out_shape=jax.ShapeDtypeStruct((n, d), jnp.float32),
    )


def kernel(node_embeddings, gc_weights, edge_index):
    n, d = node_embeddings.shape
    e = edge_index.shape[1]

    step_edges = NC * NS * CHUNK * 2
    e_pad = ((e + step_edges - 1) // step_edges) * step_edges
    steps_per_tile = e_pad // (NC * NS * CHUNK)
    blk = 512
    quantum = NS * CHUNK
    n_pad = ((n + 1 + quantum - 1) // quantum) * quantum

    rows_per_tile = n_pad // NS

    pad = e_pad - e
    src = jnp.concatenate(
        [edge_index[0], jnp.zeros((pad,), jnp.int32)]).reshape(-1, CHUNK)
    dst = jnp.concatenate(
        [edge_index[1], jnp.full((pad,), n, jnp.int32)]).reshape(-1, CHUNK)
    z = jnp.zeros((CHUNK, d), jnp.float32)
    z16 = jnp.zeros((CHUNK, DW), jnp.float32)
    ones16 = jnp.ones((CHUNK, DW), jnp.float32)

    psum, pdeg = _sc_scatter(n_pad, d, e_pad, rows_per_tile, steps_per_tile)(
        node_embeddings, src, dst, z, z16, ones16)
    return _tc_combine(n, n_pad, d, blk)(psum, psum, pdeg, pdeg, gc_weights)


# speedup vs baseline: 11.0940x; 1.8373x over previous
import functools

import jax
import jax.numpy as jnp
from jax import lax
from jax.experimental import pallas as pl
from jax.experimental.pallas import tpu as pltpu
from jax.experimental.pallas import tpu_sc as plsc

NC = 2
NS = 16
CHUNK = 64
DW = 16


def _sc_scatter(n_pad, d, e_pad, rows_per_tile, steps_per_tile):
    row_chunks = rows_per_tile // CHUNK
    mesh = plsc.VectorSubcoreMesh(core_axis_name="c", subcore_axis_name="s")

    nbuf = 2

    @functools.partial(
        pl.kernel,
        mesh=mesh,
        compiler_params=pltpu.CompilerParams(use_tc_tiling_on_sc=False),
        out_type=[
            jax.ShapeDtypeStruct((NC * n_pad, d), jnp.float32),
            jax.ShapeDtypeStruct((NC * n_pad, DW), jnp.float32),
        ],
        scratch_types=[
            pltpu.VMEM_SHARED((n_pad, d), jnp.float32),
            pltpu.VMEM_SHARED((n_pad, DW), jnp.float32),
            pltpu.VMEM((steps_per_tile, CHUNK), jnp.int32),
            pltpu.VMEM((steps_per_tile, CHUNK), jnp.int32),
            [pltpu.VMEM((CHUNK, d), jnp.float32)] * nbuf,
            pltpu.VMEM((CHUNK, DW), jnp.float32),
            [pltpu.SemaphoreType.DMA] * nbuf,
            [pltpu.SemaphoreType.DMA] * nbuf,
            [pltpu.SemaphoreType.DMA] * nbuf,
        ],
    )
    def sc_kernel(x_hbm, src_hbm, dst_hbm, z_hbm, z16_hbm, ones_hbm,
                  psum_hbm, pdeg_hbm,
                  acc, dacc, sidx, didx, rows, onesb, gsem, ssem, dsem):
        c = lax.axis_index("c")
        s = lax.axis_index("s")
        t = c * NS + s
        row0 = s * rows_per_tile

        pltpu.sync_copy(src_hbm.at[pl.ds(t * steps_per_tile, steps_per_tile)],
                        sidx)
        pltpu.sync_copy(dst_hbm.at[pl.ds(t * steps_per_tile, steps_per_tile)],
                        didx)
        pltpu.sync_copy(z_hbm, rows[0])
        pltpu.sync_copy(z16_hbm, onesb)
        zcps = []
        for j in range(row_chunks):
            zcps.append(pltpu.async_copy(
                rows[0], acc.at[pl.ds(row0 + j * CHUNK, CHUNK)],
                ssem[j % nbuf]))
            zcps.append(pltpu.async_copy(
                onesb, dacc.at[pl.ds(row0 + j * CHUNK, CHUNK)],
                dsem[j % nbuf]))
        for cp in zcps:
            cp.wait()
        pltpu.sync_copy(ones_hbm, onesb)
        plsc.subcore_barrier()

        def start_gather(b, i):
            return pltpu.async_copy(x_hbm.at[sidx.at[i]], rows[b], gsem[b])

        def wait_gather(b, i):
            pltpu.make_async_copy(x_hbm.at[sidx.at[i]], rows[b],
                                  gsem[b]).wait()

        def start_scatter(b, i):
            pltpu.async_copy(rows[b], acc.at[didx.at[i]], ssem[b], add=True)
            pltpu.async_copy(onesb, dacc.at[didx.at[i]], dsem[b], add=True)

        def wait_scatter(b, i):
            pltpu.make_async_copy(rows[b], acc.at[didx.at[i]],
                                  ssem[b]).wait()
            pltpu.make_async_copy(onesb, dacc.at[didx.at[i]],
                                  dsem[b]).wait()

        for b in range(nbuf):
            start_gather(b, b)

        @pl.loop(0, steps_per_tile // nbuf - 1)
        def step(ii):
            i = pl.multiple_of(ii * nbuf, nbuf)
            for b in range(nbuf):
                wait_gather(b, i + b)
                start_scatter(b, i + b)
            for b in range(nbuf):
                wait_scatter(b, i + b)
                start_gather(b, i + b + nbuf)

        i_last = steps_per_tile - nbuf
        for b in range(nbuf):
            wait_gather(b, i_last + b)
            start_scatter(b, i_last + b)
        for b in range(nbuf):
            wait_scatter(b, i_last + b)
        plsc.subcore_barrier()

        out0 = c * n_pad + row0
        for j in range(row_chunks):
            b = j % nbuf
            if j >= nbuf:
                pltpu.make_async_copy(
                    rows[b], psum_hbm.at[pl.ds(out0 + (j - nbuf) * CHUNK,
                                               CHUNK)], ssem[b]).wait()
            pltpu.sync_copy(acc.at[pl.ds(row0 + j * CHUNK, CHUNK)], rows[b])
            pltpu.async_copy(rows[b],
                             psum_hbm.at[pl.ds(out0 + j * CHUNK, CHUNK)],
                             ssem[b])
        for j in range(row_chunks - nbuf, row_chunks):
            b = j % nbuf
            pltpu.make_async_copy(
                rows[b], psum_hbm.at[pl.ds(out0 + j * CHUNK, CHUNK)],
                ssem[b]).wait()
        for j in range(row_chunks):
            pltpu.sync_copy(dacc.at[pl.ds(row0 + j * CHUNK, CHUNK)], onesb)
            pltpu.sync_copy(onesb,
                            pdeg_hbm.at[pl.ds(out0 + j * CHUNK, CHUNK)])

    return sc_kernel


def _tc_combine(n, n_pad, d, blk):
    def body(p0_ref, p1_ref, d0_ref, d1_ref, w_ref, o_ref):
        ssum = p0_ref[...] + p1_ref[...]
        deg = d0_ref[:, 0:1] + d1_ref[:, 0:1]
        inv = 1.0 / jnp.maximum(deg, 1.0)
        o_ref[...] = jnp.dot(ssum * inv, w_ref[...],
                             preferred_element_type=jnp.float32)

    nblk = n_pad // blk
    return pl.pallas_call(
        body,
        grid=(nblk,),
        in_specs=[
            pl.BlockSpec((blk, d), lambda i: (i, 0)),
            pl.BlockSpec((blk, d), lambda i, nblk=nblk: (nblk + i, 0)),
            pl.BlockSpec((blk, DW), lambda i: (i, 0)),
            pl.BlockSpec((blk, DW), lambda i, nblk=nblk: (nblk + i, 0)),
            pl.BlockSpec((d, d), lambda i: (0, 0)),
        ],
        out_specs=pl.BlockSpec((blk, d), lambda i: (i, 0)),
        out_shape=jax.ShapeDtypeStruct((n, d), jnp.float32),
    )


def kernel(node_embeddings, gc_weights, edge_index):
    n, d = node_embeddings.shape
    e = edge_index.shape[1]

    step_edges = NC * NS * CHUNK * 2
    e_pad = ((e + step_edges - 1) // step_edges) * step_edges
    steps_per_tile = e_pad // (NC * NS * CHUNK)
    blk = 512
    quantum = NS * CHUNK
    n_pad = ((n + 1 + quantum - 1) // quantum) * quantum

    rows_per_tile = n_pad // NS

    pad = e_pad - e
    pad_iota = jnp.arange(pad, dtype=jnp.int32)
    src = jnp.concatenate(
        [edge_index[0], pad_iota % n]).reshape(-1, CHUNK)
    dst = jnp.concatenate(
        [edge_index[1], n + pad_iota % (n_pad - n)]).reshape(-1, CHUNK)
    z = jnp.zeros((CHUNK, d), jnp.float32)
    z16 = jnp.zeros((CHUNK, DW), jnp.float32)
    ones16 = jnp.ones((CHUNK, DW), jnp.float32)

    psum, pdeg = _sc_scatter(n_pad, d, e_pad, rows_per_tile, steps_per_tile)(
        node_embeddings, src, dst, z, z16, ones16)
    return _tc_combine(n, n_pad, d, blk)(psum, psum, pdeg, pdeg, gc_weights)


# speedup vs baseline: 13.3150x; 1.2002x over previous
import functools

import jax
import jax.numpy as jnp
from jax import lax
from jax.experimental import pallas as pl
from jax.experimental.pallas import tpu as pltpu
from jax.experimental.pallas import tpu_sc as plsc

NC = 2
NS = 16
CHUNK = 64
DW = 16
NBUF = 3
SHIFT = 14


def _sc_scatter(n_pad, d, rows_per_tile, steps_per_tile):
    row_chunks = rows_per_tile // CHUNK
    mesh = plsc.VectorSubcoreMesh(core_axis_name="c", subcore_axis_name="s")

    @functools.partial(
        pl.kernel,
        mesh=mesh,
        compiler_params=pltpu.CompilerParams(use_tc_tiling_on_sc=False,
                                             needs_layout_passes=False),
        out_type=[
            jax.ShapeDtypeStruct((NC * n_pad, d), jnp.float32),
            jax.ShapeDtypeStruct((NC * n_pad,), jnp.float32),
        ],
        scratch_types=[
            pltpu.VMEM_SHARED((n_pad, d), jnp.float32),
            pltpu.VMEM_SHARED((n_pad, DW), jnp.float32),
            pltpu.VMEM((steps_per_tile, CHUNK), jnp.int32),
            [pltpu.VMEM((CHUNK,), jnp.int32)] * NBUF,
            [pltpu.VMEM((CHUNK,), jnp.int32)] * NBUF,
            [pltpu.VMEM((CHUNK, d), jnp.float32)] * NBUF,
            pltpu.VMEM((CHUNK, DW), jnp.float32),
            pltpu.VMEM((rows_per_tile,), jnp.float32),
            [pltpu.SemaphoreType.DMA] * NBUF,
            [pltpu.SemaphoreType.DMA] * NBUF,
            [pltpu.SemaphoreType.DMA] * NBUF,
        ],
    )
    def sc_kernel(x_hbm, packed_hbm, z_hbm, z16_hbm, ones_hbm,
                  psum_hbm, pdeg_hbm,
                  acc, dacc, pidx, sidx, didx, rows, onesb, degc,
                  gsem, ssem, dsem):
        c = lax.axis_index("c")
        s = lax.axis_index("s")
        t = c * NS + s
        row0 = s * rows_per_tile

        pltpu.sync_copy(
            packed_hbm.at[pl.ds(t * steps_per_tile, steps_per_tile)], pidx)
        pltpu.sync_copy(z_hbm, rows[0])
        pltpu.sync_copy(z16_hbm, onesb)
        zcps = []
        for j in range(row_chunks):
            zcps.append(pltpu.async_copy(
                rows[0], acc.at[pl.ds(row0 + j * CHUNK, CHUNK)],
                ssem[j % NBUF]))
            zcps.append(pltpu.async_copy(
                onesb, dacc.at[pl.ds(row0 + j * CHUNK, CHUNK)],
                dsem[j % NBUF]))
        for cp in zcps:
            cp.wait()
        pltpu.sync_copy(ones_hbm, onesb)
        plsc.subcore_barrier()

        def unpack(b, i):
            for g in range(CHUNK // 16):
                p = pidx[i, pl.ds(g * 16, 16)]
                sidx[b][pl.ds(g * 16, 16)] = jnp.right_shift(p, SHIFT)
                didx[b][pl.ds(g * 16, 16)] = jnp.bitwise_and(p, (1 << SHIFT) - 1)

        def start_gather(b):
            pltpu.async_copy(x_hbm.at[sidx[b]], rows[b], gsem[b])

        def wait_gather(b):
            pltpu.make_async_copy(x_hbm.at[sidx[b]], rows[b], gsem[b]).wait()

        def start_scatter(b):
            pltpu.async_copy(rows[b], acc.at[didx[b]], ssem[b], add=True)
            pltpu.async_copy(onesb, dacc.at[didx[b]], dsem[b], add=True)

        def wait_scatter(b):
            pltpu.make_async_copy(rows[b], acc.at[didx[b]], ssem[b]).wait()
            pltpu.make_async_copy(onesb, dacc.at[didx[b]], dsem[b]).wait()

        for b in range(NBUF):
            unpack(b, b)
            start_gather(b)

        @pl.loop(0, steps_per_tile // NBUF - 1)
        def step(ii):
            i = pl.multiple_of(ii * NBUF, NBUF)
            for b in range(NBUF):
                wait_gather(b)
                start_scatter(b)
            for b in range(NBUF):
                wait_scatter(b)
                unpack(b, i + b + NBUF)
                start_gather(b)

        for b in range(NBUF):
            wait_gather(b)
            start_scatter(b)
        for b in range(NBUF):
            wait_scatter(b)
        plsc.subcore_barrier()

        out0 = c * n_pad + row0
        col0 = jnp.zeros((16,), jnp.int32)
        lane = lax.iota(jnp.int32, 16)
        for j in range(row_chunks):
            b = j % NBUF
            if j >= NBUF:
                pltpu.make_async_copy(
                    rows[b], psum_hbm.at[pl.ds(out0 + (j - NBUF) * CHUNK,
                                               CHUNK)], ssem[b]).wait()
            pltpu.sync_copy(acc.at[pl.ds(row0 + j * CHUNK, CHUNK)], rows[b])
            pltpu.async_copy(rows[b],
                             psum_hbm.at[pl.ds(out0 + j * CHUNK, CHUNK)],
                             ssem[b])
            pltpu.sync_copy(dacc.at[pl.ds(row0 + j * CHUNK, CHUNK)], onesb)
            for g in range(CHUNK // 16):
                v = plsc.load_gather(onesb, [lane + g * 16, col0])
                degc[pl.ds(j * CHUNK + g * 16, 16)] = v
        for j in range(row_chunks - NBUF, row_chunks):
            b = j % NBUF
            pltpu.make_async_copy(
                rows[b], psum_hbm.at[pl.ds(out0 + j * CHUNK, CHUNK)],
                ssem[b]).wait()
        pltpu.sync_copy(degc, pdeg_hbm.at[pl.ds(out0, rows_per_tile)])

    return sc_kernel


def _tc_combine(n, n_pad, d, blk):
    def body(p0_ref, p1_ref, d0_ref, d1_ref, w_ref, o_ref):
        ssum = p0_ref[...] + p1_ref[...]
        deg = (d0_ref[...] + d1_ref[...]).reshape(blk, 1)
        inv = 1.0 / jnp.maximum(deg, 1.0)
        o_ref[...] = jnp.dot(ssum * inv, w_ref[...],
                             preferred_element_type=jnp.float32)

    nblk = n_pad // blk
    return pl.pallas_call(
        body,
        grid=(nblk,),
        in_specs=[
            pl.BlockSpec((blk, d), lambda i: (i, 0)),
            pl.BlockSpec((blk, d), lambda i, nblk=nblk: (nblk + i, 0)),
            pl.BlockSpec((blk,), lambda i: (i,)),
            pl.BlockSpec((blk,), lambda i, nblk=nblk: (nblk + i,)),
            pl.BlockSpec((d, d), lambda i: (0, 0)),
        ],
        out_specs=pl.BlockSpec((blk, d), lambda i: (i, 0)),
        out_shape=jax.ShapeDtypeStruct((n, d), jnp.float32),
    )


def kernel(node_embeddings, gc_weights, edge_index):
    n, d = node_embeddings.shape
    e = edge_index.shape[1]

    step_edges = NC * NS * CHUNK * NBUF
    e_pad = ((e + step_edges - 1) // step_edges) * step_edges
    steps_per_tile = e_pad // (NC * NS * CHUNK)
    blk = 512
    quantum = NS * CHUNK
    n_pad = ((n + 1 + quantum - 1) // quantum) * quantum

    rows_per_tile = n_pad // NS

    pad = e_pad - e
    pad_iota = jnp.arange(pad, dtype=jnp.int32)
    pad_packed = jnp.left_shift(pad_iota % n, SHIFT) | (
        n + pad_iota % (n_pad - n))
    packed = jnp.concatenate(
        [jnp.left_shift(edge_index[0], SHIFT) | edge_index[1], pad_packed]
    ).reshape(-1, CHUNK)
    z = jnp.zeros((CHUNK, d), jnp.float32)
    z16 = jnp.zeros((CHUNK, DW), jnp.float32)
    ones16 = jnp.ones((CHUNK, DW), jnp.float32)

    psum, pdeg = _sc_scatter(n_pad, d, rows_per_tile, steps_per_tile)(
        node_embeddings, packed, z, z16, ones16)
    return _tc_combine(n, n_pad, d, blk)(psum, psum, pdeg, pdeg, gc_weights)


# speedup vs baseline: 13.7250x; 1.0308x over previous
import functools

import jax
import jax.numpy as jnp
from jax import lax
from jax.experimental import pallas as pl
from jax.experimental.pallas import tpu as pltpu
from jax.experimental.pallas import tpu_sc as plsc

NC = 2
NS = 16
CHUNK = 64
DW = 16
NBUF = 3
SHIFT = 14


def _sc_scatter(n_pad, d, rows_per_tile, steps_per_tile):
    row_chunks = rows_per_tile // CHUNK
    mesh = plsc.VectorSubcoreMesh(core_axis_name="c", subcore_axis_name="s")

    @functools.partial(
        pl.kernel,
        mesh=mesh,
        compiler_params=pltpu.CompilerParams(use_tc_tiling_on_sc=False,
                                             needs_layout_passes=False),
        out_type=[
            jax.ShapeDtypeStruct((NC * n_pad, d), jnp.float32),
            jax.ShapeDtypeStruct((NC * n_pad,), jnp.float32),
        ],
        scratch_types=[
            pltpu.VMEM_SHARED((n_pad, d), jnp.float32),
            pltpu.VMEM_SHARED((n_pad, DW), jnp.float32),
            pltpu.VMEM((steps_per_tile, CHUNK), jnp.int32),
            [pltpu.VMEM((CHUNK,), jnp.int32)] * NBUF,
            [pltpu.VMEM((CHUNK,), jnp.int32)] * NBUF,
            [pltpu.VMEM((CHUNK, d), jnp.float32)] * NBUF,
            pltpu.VMEM((CHUNK, DW), jnp.float32),
            pltpu.VMEM((rows_per_tile,), jnp.float32),
            [pltpu.SemaphoreType.DMA] * NBUF,
            [pltpu.SemaphoreType.DMA] * NBUF,
            [pltpu.SemaphoreType.DMA] * NBUF,
        ],
    )
    def sc_kernel(x_hbm, packed_hbm, z_hbm, z16_hbm, ones_hbm,
                  psum_hbm, pdeg_hbm,
                  acc, dacc, pidx, sidx, didx, rows, onesb, degc,
                  gsem, ssem, dsem):
        c = lax.axis_index("c")
        s = lax.axis_index("s")
        t = c * NS + s
        row0 = s * rows_per_tile

        pltpu.sync_copy(
            packed_hbm.at[pl.ds(t * steps_per_tile, steps_per_tile)], pidx)
        pltpu.sync_copy(z_hbm, rows[0])
        pltpu.sync_copy(z16_hbm, onesb)
        zcps = []
        for j in range(row_chunks):
            zcps.append(pltpu.async_copy(
                rows[0], acc.at[pl.ds(row0 + j * CHUNK, CHUNK)],
                ssem[j % NBUF]))
            zcps.append(pltpu.async_copy(
                onesb, dacc.at[pl.ds(row0 + j * CHUNK, CHUNK)],
                dsem[j % NBUF]))
        for cp in zcps:
            cp.wait()
        pltpu.sync_copy(ones_hbm, onesb)
        plsc.subcore_barrier()

        def unpack(b, i):
            for g in range(CHUNK // 16):
                p = pidx[i, pl.ds(g * 16, 16)]
                sidx[b][pl.ds(g * 16, 16)] = jnp.right_shift(p, SHIFT)
                didx[b][pl.ds(g * 16, 16)] = jnp.bitwise_and(p, (1 << SHIFT) - 1)

        def start_gather(b):
            pltpu.async_copy(x_hbm.at[sidx[b]], rows[b], gsem[b])

        def wait_gather(b):
            pltpu.make_async_copy(x_hbm.at[sidx[b]], rows[b], gsem[b]).wait()

        def start_scatter(b):
            pltpu.async_copy(rows[b], acc.at[didx[b]], ssem[b], add=True)
            pltpu.async_copy(onesb, dacc.at[didx[b]], dsem[b], add=True)

        def wait_scatter(b):
            pltpu.make_async_copy(rows[b], acc.at[didx[b]], ssem[b]).wait()
            pltpu.make_async_copy(onesb, dacc.at[didx[b]], dsem[b]).wait()

        for b in range(NBUF):
            unpack(b, b)
            start_gather(b)

        @pl.loop(0, steps_per_tile // NBUF - 1)
        def step(ii):
            i = pl.multiple_of(ii * NBUF, NBUF)
            for b in range(NBUF):
                wait_gather(b)
                start_scatter(b)
            for b in range(NBUF):
                wait_scatter(b)
                unpack(b, i + b + NBUF)
                start_gather(b)

        for b in range(NBUF):
            wait_gather(b)
            start_scatter(b)
        for b in range(NBUF):
            wait_scatter(b)
        plsc.subcore_barrier()

        out0 = c * n_pad + row0
        col0 = jnp.zeros((16,), jnp.int32)
        lane = lax.iota(jnp.int32, 16)
        for j in range(row_chunks):
            b = j % NBUF
            if j >= NBUF:
                pltpu.make_async_copy(
                    rows[b], psum_hbm.at[pl.ds(out0 + (j - NBUF) * CHUNK,
                                               CHUNK)], ssem[b]).wait()
            pltpu.sync_copy(acc.at[pl.ds(row0 + j * CHUNK, CHUNK)], rows[b])
            pltpu.async_copy(rows[b],
                             psum_hbm.at[pl.ds(out0 + j * CHUNK, CHUNK)],
                             ssem[b])
            pltpu.sync_copy(dacc.at[pl.ds(row0 + j * CHUNK, CHUNK)], onesb)
            for g in range(CHUNK // 16):
                v = plsc.load_gather(onesb, [lane + g * 16, col0])
                degc[pl.ds(j * CHUNK + g * 16, 16)] = v
        for j in range(row_chunks - NBUF, row_chunks):
            b = j % NBUF
            pltpu.make_async_copy(
                rows[b], psum_hbm.at[pl.ds(out0 + j * CHUNK, CHUNK)],
                ssem[b]).wait()
        pltpu.sync_copy(degc, pdeg_hbm.at[pl.ds(out0, rows_per_tile)])

    return sc_kernel


def _tc_combine(n, n_pad, d, blk):
    def body(p0_ref, p1_ref, d0_ref, d1_ref, w_ref, o_ref):
        ssum = p0_ref[...] + p1_ref[...]
        deg = (d0_ref[...] + d1_ref[...]).reshape(blk, 1)
        inv = 1.0 / jnp.maximum(deg, 1.0)
        o_ref[...] = jnp.dot(ssum * inv, w_ref[...],
                             preferred_element_type=jnp.float32)

    nblk = n_pad // blk
    return pl.pallas_call(
        body,
        grid=(nblk,),
        in_specs=[
            pl.BlockSpec((blk, d), lambda i: (i, 0)),
            pl.BlockSpec((blk, d), lambda i, nblk=nblk: (nblk + i, 0)),
            pl.BlockSpec((blk,), lambda i: (i,)),
            pl.BlockSpec((blk,), lambda i, nblk=nblk: (nblk + i,)),
            pl.BlockSpec((d, d), lambda i: (0, 0)),
        ],
        out_specs=pl.BlockSpec((blk, d), lambda i: (i, 0)),
        out_shape=jax.ShapeDtypeStruct((n, d), jnp.float32),
    )


def kernel(node_embeddings, gc_weights, edge_index):
    n, d = node_embeddings.shape
    e = edge_index.shape[1]

    step_edges = NC * NS * CHUNK * NBUF
    e_pad = ((e + step_edges - 1) // step_edges) * step_edges
    steps_per_tile = e_pad // (NC * NS * CHUNK)
    blk = 1024
    quantum = NS * CHUNK
    n_pad = ((n + 1 + quantum - 1) // quantum) * quantum

    rows_per_tile = n_pad // NS

    pad = e_pad - e
    pad_iota = jnp.arange(pad, dtype=jnp.int32)
    pad_packed = jnp.left_shift(pad_iota % n, SHIFT) | (
        n + pad_iota % (n_pad - n))
    packed = jnp.concatenate(
        [jnp.left_shift(edge_index[0], SHIFT) | edge_index[1], pad_packed]
    ).reshape(-1, CHUNK)
    z = jnp.zeros((CHUNK, d), jnp.float32)
    z16 = jnp.zeros((CHUNK, DW), jnp.float32)
    ones16 = jnp.ones((CHUNK, DW), jnp.float32)

    psum, pdeg = _sc_scatter(n_pad, d, rows_per_tile, steps_per_tile)(
        node_embeddings, packed, z, z16, ones16)
    return _tc_combine(n, n_pad, d, blk)(psum, psum, pdeg, pdeg, gc_weights)


# speedup vs baseline: 14.1261x; 1.0292x over previous
import functools

import jax
import jax.numpy as jnp
from jax import lax
from jax.experimental import pallas as pl
from jax.experimental.pallas import tpu as pltpu
from jax.experimental.pallas import tpu_sc as plsc

NC = 2
NS = 16
CHUNK = 64
DW = 16
NBUF = 4
SHIFT = 14


def _sc_scatter(n_pad, d, rows_per_tile, steps_per_tile):
    row_chunks = rows_per_tile // CHUNK
    mesh = plsc.VectorSubcoreMesh(core_axis_name="c", subcore_axis_name="s")

    @functools.partial(
        pl.kernel,
        mesh=mesh,
        compiler_params=pltpu.CompilerParams(use_tc_tiling_on_sc=False,
                                             needs_layout_passes=False),
        out_type=[
            jax.ShapeDtypeStruct((NC * n_pad, d), jnp.float32),
            jax.ShapeDtypeStruct((NC * n_pad,), jnp.float32),
        ],
        scratch_types=[
            pltpu.VMEM_SHARED((n_pad, d), jnp.float32),
            pltpu.VMEM_SHARED((n_pad, DW), jnp.float32),
            [pltpu.VMEM((CHUNK,), jnp.int32)] * NBUF,
            [pltpu.VMEM((CHUNK,), jnp.int32)] * NBUF,
            [pltpu.VMEM((CHUNK,), jnp.int32)] * NBUF,
            [pltpu.VMEM((CHUNK, d), jnp.float32)] * NBUF,
            pltpu.VMEM((CHUNK, DW), jnp.float32),
            pltpu.VMEM((rows_per_tile,), jnp.float32),
            [pltpu.SemaphoreType.DMA] * NBUF,
            [pltpu.SemaphoreType.DMA] * NBUF,
            [pltpu.SemaphoreType.DMA] * NBUF,
            [pltpu.SemaphoreType.DMA] * NBUF,
        ],
    )
    def sc_kernel(x_hbm, packed_hbm, z_hbm, z16_hbm, ones_hbm,
                  psum_hbm, pdeg_hbm,
                  acc, dacc, pidx, sidx, didx, rows, onesb, degc,
                  isem, gsem, ssem, dsem):
        c = lax.axis_index("c")
        s = lax.axis_index("s")
        t = c * NS + s
        row0 = s * rows_per_tile

        cbase = t * steps_per_tile
        last = steps_per_tile - 1
        pltpu.sync_copy(z_hbm, rows[0])
        pltpu.sync_copy(z16_hbm, onesb)
        zcps = []
        for j in range(row_chunks):
            zcps.append(pltpu.async_copy(
                rows[0], acc.at[pl.ds(row0 + j * CHUNK, CHUNK)],
                ssem[j % NBUF]))
            zcps.append(pltpu.async_copy(
                onesb, dacc.at[pl.ds(row0 + j * CHUNK, CHUNK)],
                dsem[j % NBUF]))
        for cp in zcps:
            cp.wait()
        pltpu.sync_copy(ones_hbm, onesb)
        plsc.subcore_barrier()

        def start_idx(b, i):
            ci = cbase + jnp.minimum(i, last)
            pltpu.async_copy(packed_hbm.at[ci], pidx[b], isem[b])

        def wait_idx(b, i):
            ci = cbase + jnp.minimum(i, last)
            pltpu.make_async_copy(packed_hbm.at[ci], pidx[b], isem[b]).wait()

        def unpack(b):
            for g in range(CHUNK // 16):
                p = pidx[b][pl.ds(g * 16, 16)]
                sidx[b][pl.ds(g * 16, 16)] = jnp.right_shift(p, SHIFT)
                didx[b][pl.ds(g * 16, 16)] = jnp.bitwise_and(p, (1 << SHIFT) - 1)

        def start_gather(b):
            pltpu.async_copy(x_hbm.at[sidx[b]], rows[b], gsem[b])

        def wait_gather(b):
            pltpu.make_async_copy(x_hbm.at[sidx[b]], rows[b], gsem[b]).wait()

        def start_scatter(b):
            pltpu.async_copy(rows[b], acc.at[didx[b]], ssem[b], add=True)
            pltpu.async_copy(onesb, dacc.at[didx[b]], dsem[b], add=True)

        def wait_scatter(b):
            pltpu.make_async_copy(rows[b], acc.at[didx[b]], ssem[b]).wait()
            pltpu.make_async_copy(onesb, dacc.at[didx[b]], dsem[b]).wait()

        for b in range(NBUF):
            start_idx(b, b)
        for b in range(NBUF):
            wait_idx(b, b)
            unpack(b)
            start_idx(b, b + NBUF)
            start_gather(b)

        @pl.loop(0, steps_per_tile // NBUF - 1)
        def step(ii):
            i = pl.multiple_of(ii * NBUF, NBUF)
            for b in range(NBUF):
                wait_gather(b)
                start_scatter(b)
            for b in range(NBUF):
                wait_scatter(b)
                wait_idx(b, i + b + NBUF)
                unpack(b)
                start_idx(b, i + b + 2 * NBUF)
                start_gather(b)

        for b in range(NBUF):
            wait_gather(b)
            start_scatter(b)
        for b in range(NBUF):
            wait_scatter(b)
            wait_idx(b, steps_per_tile + b)
        plsc.subcore_barrier()

        out0 = c * n_pad + row0
        col0 = jnp.zeros((16,), jnp.int32)
        lane = lax.iota(jnp.int32, 16)
        for j in range(row_chunks):
            b = j % NBUF
            if j >= NBUF:
                pltpu.make_async_copy(
                    rows[b], psum_hbm.at[pl.ds(out0 + (j - NBUF) * CHUNK,
                                               CHUNK)], ssem[b]).wait()
            pltpu.sync_copy(acc.at[pl.ds(row0 + j * CHUNK, CHUNK)], rows[b])
            pltpu.async_copy(rows[b],
                             psum_hbm.at[pl.ds(out0 + j * CHUNK, CHUNK)],
                             ssem[b])
            pltpu.sync_copy(dacc.at[pl.ds(row0 + j * CHUNK, CHUNK)], onesb)
            for g in range(CHUNK // 16):
                v = plsc.load_gather(onesb, [lane + g * 16, col0])
                degc[pl.ds(j * CHUNK + g * 16, 16)] = v
        for j in range(row_chunks - NBUF, row_chunks):
            b = j % NBUF
            pltpu.make_async_copy(
                rows[b], psum_hbm.at[pl.ds(out0 + j * CHUNK, CHUNK)],
                ssem[b]).wait()
        pltpu.sync_copy(degc, pdeg_hbm.at[pl.ds(out0, rows_per_tile)])

    return sc_kernel


def _tc_combine(n, n_pad, d, blk):
    def body(p0_ref, p1_ref, d0_ref, d1_ref, w_ref, o_ref):
        ssum = p0_ref[...] + p1_ref[...]
        deg = (d0_ref[...] + d1_ref[...]).reshape(blk, 1)
        inv = 1.0 / jnp.maximum(deg, 1.0)
        o_ref[...] = jnp.dot(ssum * inv, w_ref[...],
                             preferred_element_type=jnp.float32)

    nblk = n_pad // blk
    return pl.pallas_call(
        body,
        grid=(nblk,),
        in_specs=[
            pl.BlockSpec((blk, d), lambda i: (i, 0)),
            pl.BlockSpec((blk, d), lambda i, nblk=nblk: (nblk + i, 0)),
            pl.BlockSpec((blk,), lambda i: (i,)),
            pl.BlockSpec((blk,), lambda i, nblk=nblk: (nblk + i,)),
            pl.BlockSpec((d, d), lambda i: (0, 0)),
        ],
        out_specs=pl.BlockSpec((blk, d), lambda i: (i, 0)),
        out_shape=jax.ShapeDtypeStruct((n, d), jnp.float32),
    )


def kernel(node_embeddings, gc_weights, edge_index):
    n, d = node_embeddings.shape
    e = edge_index.shape[1]

    step_edges = NC * NS * CHUNK * NBUF
    e_pad = ((e + step_edges - 1) // step_edges) * step_edges
    steps_per_tile = e_pad // (NC * NS * CHUNK)
    blk = 1024
    quantum = NS * CHUNK
    n_pad = ((n + 1 + quantum - 1) // quantum) * quantum

    rows_per_tile = n_pad // NS

    pad = e_pad - e
    pad_iota = jnp.arange(pad, dtype=jnp.int32)
    pad_packed = jnp.left_shift(pad_iota % n, SHIFT) | (
        n + pad_iota % (n_pad - n))
    packed = jnp.concatenate(
        [jnp.left_shift(edge_index[0], SHIFT) | edge_index[1], pad_packed]
    ).reshape(-1, CHUNK)
    z = jnp.zeros((CHUNK, d), jnp.float32)
    z16 = jnp.zeros((CHUNK, DW), jnp.float32)
    ones16 = jnp.ones((CHUNK, DW), jnp.float32)

    psum, pdeg = _sc_scatter(n_pad, d, rows_per_tile, steps_per_tile)(
        node_embeddings, packed, z, z16, ones16)
    return _tc_combine(n, n_pad, d, blk)(psum, psum, pdeg, pdeg, gc_weights)
